# baseline (speedup 1.0000x reference)
import functools

import jax
import jax.numpy as jnp
from jax import lax
from jax.experimental import pallas as pl
from jax.experimental.pallas import tpu as pltpu
from jax.experimental.pallas import tpu_sc as plsc

N = 10000
E = 320000
NC = 2
NS = 16
K = 128
EPT = 20480
EPAD = EPT * NS
NPAD = 10240
RPT = NPAD // NS
NRT = N // NS
NBUF = 4
NSLOT = 8
LOOK = 5
KGRP = 8
HFRAC = 0

K1 = 64
NCH1 = EPT // K1
BLK = 40
NBLK = NCH1 // BLK
NCH2 = EPT // K
NCHD = NCH2 // NC

_MESH = plsc.VectorSubcoreMesh(core_axis_name="c", subcore_axis_name="s")


def _zero_vmem_2d(buf, rows, cols):
    z = jnp.zeros((16,), jnp.float32)

    def row(r, _):
        def col(c, __):
            buf[r, pl.ds(c * 16, 16)] = z
            return 0
        return lax.fori_loop(0, cols // 16, col, 0)

    lax.fori_loop(0, rows, row, 0)


def _zero_vmem_1d(buf, n):
    z = jnp.zeros((16,), jnp.float32)

    def col(c, _):
        buf[pl.ds(c * 16, 16)] = z
        return 0

    lax.fori_loop(0, n // 16, col, 0)


@functools.partial(
    pl.kernel,
    out_type=jax.ShapeDtypeStruct((NC, NPAD), jnp.float32),
    mesh=_MESH,
    scratch_types=[
        pltpu.VMEM((NCH2, K), jnp.int32),
        pltpu.VMEM((K,), jnp.float32),
        pltpu.VMEM((RPT,), jnp.float32),
        pltpu.VMEM_SHARED((NPAD,), jnp.float32),
        pltpu.SemaphoreType.DMA,
    ],
)
def _deg_kernel(es_hbm, out_hbm, didx, ones_v, obuf, acc, sem):
    cid = lax.axis_index("c")
    sid = lax.axis_index("s")

    pltpu.sync_copy(es_hbm.at[1, sid], didx)

    def fill(c, _):
        ones_v[pl.ds(c * 16, 16)] = jnp.ones((16,), jnp.float32)
        return 0
    lax.fori_loop(0, K // 16, fill, 0)
    _zero_vmem_1d(obuf, RPT)
    pltpu.sync_copy(obuf, acc.at[pl.ds(sid * RPT, RPT)])
    plsc.subcore_barrier()

    def grp(g, _):
        descs = [
            pltpu.async_copy(
                ones_v, acc.at[didx.at[cid * NCHD + g * KGRP + b]],
                sem, add=True)
            for b in range(KGRP)
        ]
        for d in descs:
            d.wait()
        return 0
    lax.fori_loop(0, NCHD // KGRP, grp, 0)
    plsc.subcore_barrier()

    pltpu.sync_copy(acc.at[pl.ds(sid * RPT, RPT)], obuf)
    pltpu.sync_copy(obuf, out_hbm.at[cid, pl.ds(sid * RPT, RPT)])


@functools.partial(
    pl.kernel,
    out_type=jax.ShapeDtypeStruct((NC, NPAD, 64), jnp.float32),
    mesh=_MESH,
    compiler_params=pltpu.CompilerParams(use_tc_tiling_on_sc=False),
    scratch_types=[
        pltpu.VMEM((BLK, K1), jnp.int32),
        pltpu.VMEM((BLK, K1), jnp.int32),
        pltpu.VMEM((BLK, K1), jnp.int32),
        pltpu.VMEM((BLK, K1), jnp.int32),
        *[pltpu.VMEM((K1, 64), jnp.float32) for _ in range(NSLOT)],
        pltpu.VMEM_SHARED((N, 64), jnp.float32),
        pltpu.VMEM_SHARED((NPAD, 64), jnp.float32),
        pltpu.SemaphoreType.DMA,
        pltpu.SemaphoreType.DMA,
        *[pltpu.SemaphoreType.DMA for _ in range(NSLOT)],
        *[pltpu.SemaphoreType.DMA for _ in range(NSLOT)],
        pltpu.SemaphoreType.DMA,
    ],
)
def _agg128(h0_hbm, h1_hbm, es1_hbm, out_hbm, sb0, sb1, db0, db1,
            r0, r1, r2, r3, r4, r5, r6, r7, htab, acc, i0, i1,
            g0, g1, g2, g3, g4, g5, g6, g7,
            c0, c1, c2, c3, c4, c5, c6, c7, tsem):
    rows = (r0, r1, r2, r3, r4, r5, r6, r7)
    gsem = (g0, g1, g2, g3, g4, g5, g6, g7)
    ssem = (c0, c1, c2, c3, c4, c5, c6, c7)
    sbuf = (sb0, sb1)
    dbuf = (db0, db1)
    isem = (i0, i1)
    cid = lax.axis_index("c")
    sid = lax.axis_index("s")

    def gather(sb, l, b):
        pltpu.async_copy(htab.at[sb.at[l]], rows[b], gsem[b])

    def gwait(sb, l, b):
        pltpu.make_async_copy(htab.at[sb.at[l]], rows[b], gsem[b]).wait()

    def scat(db, l, b):
        pltpu.async_copy(rows[b], acc.at[db.at[l]], ssem[b], add=True)

    def swait(b):
        pltpu.make_async_copy(rows[b], acc.at[dbuf[0].at[0]], ssem[b]).wait()

    @pl.when(cid == 0)
    def _():
        pltpu.async_copy(h0_hbm.at[pl.ds(sid * NRT, NRT)],
                         htab.at[pl.ds(sid * NRT, NRT)], tsem)

    @pl.when(cid != 0)
    def _():
        pltpu.async_copy(h1_hbm.at[pl.ds(sid * NRT, NRT)],
                         htab.at[pl.ds(sid * NRT, NRT)], tsem)

    pltpu.async_copy(es1_hbm.at[0, sid, pl.ds(0, BLK)], sbuf[0], isem[0])
    pltpu.async_copy(es1_hbm.at[1, sid, pl.ds(0, BLK)], dbuf[0], isem[0])

    _zero_vmem_2d(rows[0], 64, 64)

    def zc(c, _):
        pltpu.sync_copy(rows[0], acc.at[pl.ds(sid * RPT + c * 64, 64)])
        return 0
    lax.fori_loop(0, RPT // 64, zc, 0)

    pltpu.make_async_copy(
        es1_hbm.at[0, sid, pl.ds(0, BLK)], sbuf[0], isem[0]).wait()
    pltpu.make_async_copy(
        es1_hbm.at[1, sid, pl.ds(0, BLK)], dbuf[0], isem[0]).wait()
    pltpu.make_async_copy(
        h0_hbm.at[pl.ds(sid * NRT, NRT)], htab.at[pl.ds(sid * NRT, NRT)],
        tsem).wait()
    plsc.subcore_barrier()

    for blk in range(NBLK):
        p = blk % 2
        sb, db = sbuf[p], dbuf[p]

        if blk == 0:
            for b in range(LOOK):
                gather(sb, jnp.int32(b), b)
            for b in range(NSLOT):
                gwait(sb, jnp.int32(b), b)
                scat(db, jnp.int32(b), b)
                if b >= NSLOT - LOOK:
                    swait((b + LOOK) % NSLOT)
                gather(sb, jnp.int32(b + LOOK), (b + LOOK) % NSLOT)
            g_start = 1
        else:
            g_start = 0

        def grp(g, _):
            for b in range(NSLOT):
                l = g * NSLOT + b
                c = (b + LOOK) % NSLOT
                gwait(sb, l, b)
                scat(db, l, b)
                swait(c)
                gather(sb, l + LOOK, c)
            return 0
        lax.fori_loop(g_start, (BLK - NSLOT) // NSLOT, grp, 0)

        if blk + 1 < NBLK:
            pltpu.async_copy(es1_hbm.at[0, sid, pl.ds((blk + 1) * BLK, BLK)],
                             sbuf[1 - p], isem[1 - p])
            pltpu.async_copy(es1_hbm.at[1, sid, pl.ds((blk + 1) * BLK, BLK)],
                             dbuf[1 - p], isem[1 - p])

        for b in range(NSLOT):
            l = BLK - NSLOT + b
            c = (b + LOOK) % NSLOT
            gwait(sb, jnp.int32(l), b)
            scat(db, jnp.int32(l), b)
            swait(c)
            if b < NSLOT - LOOK:
                gather(sb, jnp.int32(l + LOOK), c)
            elif blk + 1 < NBLK:
                if b == NSLOT - LOOK:
                    pltpu.make_async_copy(
                        es1_hbm.at[0, sid, pl.ds((blk + 1) * BLK, BLK)],
                        sbuf[1 - p], isem[1 - p]).wait()
                    pltpu.make_async_copy(
                        es1_hbm.at[1, sid, pl.ds((blk + 1) * BLK, BLK)],
                        dbuf[1 - p], isem[1 - p]).wait()
                gather(sbuf[1 - p], jnp.int32(b - (NSLOT - LOOK)), c)

    for b in range(LOOK, NSLOT):
        swait(b)
    plsc.subcore_barrier()

    def oc(c, _):
        r0_ = sid * RPT + c * 64
        pltpu.sync_copy(acc.at[pl.ds(r0_, 64)], rows[0])
        pltpu.sync_copy(rows[0], out_hbm.at[cid, pl.ds(r0_, 64)])
        return 0
    lax.fori_loop(0, RPT // 64, oc, 0)


@functools.partial(
    pl.kernel,
    out_type=jax.ShapeDtypeStruct((NC, NPAD, 32), jnp.float32),
    mesh=_MESH,
    compiler_params=pltpu.CompilerParams(use_tc_tiling_on_sc=False),
    scratch_types=[
        pltpu.VMEM((NCH2, K), jnp.int32),
        pltpu.VMEM((NCH2, K), jnp.int32),
        *[pltpu.VMEM((K, 32), jnp.float32) for _ in range(NSLOT)],
        pltpu.VMEM((128, 32), jnp.float32),
        pltpu.VMEM_SHARED((N, 32), jnp.float32),
        pltpu.VMEM_SHARED((NPAD, 32), jnp.float32),
        *[pltpu.SemaphoreType.DMA for _ in range(NSLOT)],
        *[pltpu.SemaphoreType.DMA for _ in range(NSLOT)],
        pltpu.SemaphoreType.DMA,
    ],
)
def _agg64(h0_hbm, h1_hbm, es2_hbm, out_hbm, sidx, didx,
           r0, r1, r2, r3, r4, r5, r6, r7,
           obuf, htab, acc,
           g0, g1, g2, g3, g4, g5, g6, g7,
           c0, c1, c2, c3, c4, c5, c6, c7, tsem):
    rows = (r0, r1, r2, r3, r4, r5, r6, r7)
    gsem = (g0, g1, g2, g3, g4, g5, g6, g7)
    ssem = (c0, c1, c2, c3, c4, c5, c6, c7)
    cid = lax.axis_index("c")
    sid = lax.axis_index("s")

    def issue_gather(i, b):
        m = lax.rem(i, 16)

        @pl.when(jnp.logical_and(m < HFRAC, cid == 0))
        def _():
            pltpu.async_copy(h0_hbm.at[sidx.at[i]], rows[b], gsem[b])

        @pl.when(jnp.logical_and(m < HFRAC, cid != 0))
        def _():
            pltpu.async_copy(h1_hbm.at[sidx.at[i]], rows[b], gsem[b])

        @pl.when(m >= HFRAC)
        def _():
            pltpu.async_copy(htab.at[sidx.at[i]], rows[b], gsem[b])

    @pl.when(cid == 0)
    def _():
        pltpu.async_copy(h0_hbm.at[pl.ds(sid * NRT, NRT)],
                         htab.at[pl.ds(sid * NRT, NRT)], tsem)

    @pl.when(cid != 0)
    def _():
        pltpu.async_copy(h1_hbm.at[pl.ds(sid * NRT, NRT)],
                         htab.at[pl.ds(sid * NRT, NRT)], tsem)

    pltpu.sync_copy(es2_hbm.at[0, sid], sidx)
    pltpu.sync_copy(es2_hbm.at[1, sid], didx)

    _zero_vmem_2d(obuf, 128, 32)

    def zc(c, _):
        pltpu.sync_copy(obuf, acc.at[pl.ds(sid * RPT + c * 128, 128)])
        return 0
    lax.fori_loop(0, RPT // 128, zc, 0)

    pltpu.make_async_copy(
        h0_hbm.at[pl.ds(sid * NRT, NRT)], htab.at[pl.ds(sid * NRT, NRT)],
        tsem).wait()
    plsc.subcore_barrier()

    for b in range(LOOK):
        issue_gather(jnp.int32(b), b)

    for b in range(NSLOT):
        c = (b + LOOK) % NSLOT
        pltpu.make_async_copy(htab.at[sidx.at[b]], rows[b], gsem[b]).wait()
        pltpu.async_copy(rows[b], acc.at[didx.at[b]], ssem[b], add=True)
        if b >= NSLOT - LOOK:
            pltpu.make_async_copy(rows[c], acc.at[didx.at[b]], ssem[c]).wait()
        issue_gather(jnp.int32(b + LOOK), c)

    def grp(g, _):
        for b in range(NSLOT):
            i = g * NSLOT + b
            c = (b + LOOK) % NSLOT
            pltpu.make_async_copy(
                htab.at[sidx.at[i]], rows[b], gsem[b]).wait()
            pltpu.async_copy(rows[b], acc.at[didx.at[i]], ssem[b], add=True)
            pltpu.make_async_copy(rows[c], acc.at[didx.at[i]], ssem[c]).wait()

            @pl.when(i + LOOK < NCH2)
            def _():
                issue_gather(i + LOOK, c)
        return 0
    lax.fori_loop(1, NCH2 // NSLOT, grp, 0)
    for b in range(LOOK, NSLOT):
        pltpu.make_async_copy(rows[b], acc.at[didx.at[0]], ssem[b]).wait()
    plsc.subcore_barrier()

    def oc(c, _):
        r0_ = sid * RPT + c * 128
        pltpu.sync_copy(acc.at[pl.ds(r0_, 128)], obuf)
        pltpu.sync_copy(obuf, out_hbm.at[cid, pl.ds(r0_, 128)])
        return 0
    lax.fori_loop(0, RPT // 128, oc, 0)


def _mm0_body(x_ref, w_ref, o_ref):
    o_ref[...] = jnp.dot(x_ref[...], w_ref[...],
                         preferred_element_type=jnp.float32)


def _mm1_body(h_ref, d0_ref, d1_ref, oa_ref, ob_ref):
    dinv = lax.rsqrt(d0_ref[...] + d1_ref[...] + 1.0)
    hp = h_ref[...] * dinv
    oa_ref[...] = hp[:, :64]
    ob_ref[...] = hp[:, 64:]


def _mm2_body(p_ref, ha_ref, hb_ref, d0_ref, d1_ref, b_ref, w_ref,
              oa_ref, ob_ref):
    dinv = lax.rsqrt(d0_ref[...] + d1_ref[...] + 1.0)
    sl = p_ref[0, :N] + ha_ref[...]
    sr = p_ref[1, :N] + hb_ref[...]
    s = jnp.concatenate((sl, sr), axis=1)
    z = jnp.maximum(s * dinv + b_ref[...], 0.0)
    h = jnp.dot(z, w_ref[...], preferred_element_type=jnp.float32)
    hp = h * dinv
    oa_ref[...] = hp[:, :32]
    ob_ref[...] = hp[:, 32:]


def _out_body(q_ref, ha_ref, hb_ref, d0_ref, d1_ref, b_ref, o_ref):
    dinv = lax.rsqrt(d0_ref[...] + d1_ref[...] + 1.0)
    sl = q_ref[0, :N] + ha_ref[...]
    sr = q_ref[1, :N] + hb_ref[...]
    s = jnp.concatenate((sl, sr), axis=1)
    o_ref[...] = s * dinv + b_ref[...]


_mm0 = pl.pallas_call(
    _mm0_body, out_shape=jax.ShapeDtypeStruct((N, 128), jnp.float32))
_mm1 = pl.pallas_call(
    _mm1_body,
    out_shape=(jax.ShapeDtypeStruct((N, 64), jnp.float32),
               jax.ShapeDtypeStruct((N, 64), jnp.float32)))
_mm2 = pl.pallas_call(
    _mm2_body,
    out_shape=(jax.ShapeDtypeStruct((N, 32), jnp.float32),
               jax.ShapeDtypeStruct((N, 32), jnp.float32)))
_mm3 = pl.pallas_call(
    _out_body, out_shape=jax.ShapeDtypeStruct((N, 64), jnp.float32))


def kernel(x, edge_index, W1, b1, W2, b2):
    pad = EPAD - E
    sd = jnp.concatenate(
        [edge_index,
         jnp.stack([jnp.zeros((pad,), jnp.int32),
                    jnp.full((pad,), NPAD - 1, jnp.int32)])], axis=1)
    es1 = sd.reshape(2, NS, NCH1, K1)
    es2 = sd.reshape(2, NS, NCH2, K)

    h1 = _mm0(x, W1)
    degp = _deg_kernel(es2)
    d0 = degp[0, :N].reshape(N, 1)
    d1 = degp[1, :N].reshape(N, 1)

    h1a, h1b = _mm1(h1, d0, d1)
    p = _agg128(h1a, h1b, es1)
    h2a, h2b = _mm2(p, h1a, h1b, d0, d1, b1.reshape(1, 128), W2)
    q = _agg64(h2a, h2b, es2)
    return _mm3(q, h2a, h2b, d0, d1, b2.reshape(1, 64))


# speedup vs baseline: 1.0015x; 1.0015x over previous
import functools

import jax
import jax.numpy as jnp
from jax import lax
from jax.experimental import pallas as pl
from jax.experimental.pallas import tpu as pltpu
from jax.experimental.pallas import tpu_sc as plsc

N = 10000
E = 320000
NC = 2
NS = 16
K = 128
EPT = 20480
EPAD = EPT * NS
NPAD = 10240
RPT = NPAD // NS
NRT = N // NS
NBUF = 4
NSLOT = 8
LOOK = 4
KGRP = 8
HFRAC = 0

K1 = 64
NCH1 = EPT // K1
BLK = 40
NBLK = NCH1 // BLK
NCH2 = EPT // K
NCHD = NCH2 // NC

_MESH = plsc.VectorSubcoreMesh(core_axis_name="c", subcore_axis_name="s")


def _zero_vmem_2d(buf, rows, cols):
    z = jnp.zeros((16,), jnp.float32)

    def row(r, _):
        def col(c, __):
            buf[r, pl.ds(c * 16, 16)] = z
            return 0
        return lax.fori_loop(0, cols // 16, col, 0)

    lax.fori_loop(0, rows, row, 0)


def _zero_vmem_1d(buf, n):
    z = jnp.zeros((16,), jnp.float32)

    def col(c, _):
        buf[pl.ds(c * 16, 16)] = z
        return 0

    lax.fori_loop(0, n // 16, col, 0)


@functools.partial(
    pl.kernel,
    out_type=jax.ShapeDtypeStruct((NC, NPAD), jnp.float32),
    mesh=_MESH,
    scratch_types=[
        pltpu.VMEM((NCH2, K), jnp.int32),
        pltpu.VMEM((K,), jnp.float32),
        pltpu.VMEM((RPT,), jnp.float32),
        pltpu.VMEM_SHARED((NPAD,), jnp.float32),
        pltpu.SemaphoreType.DMA,
    ],
)
def _deg_kernel(es_hbm, out_hbm, didx, ones_v, obuf, acc, sem):
    cid = lax.axis_index("c")
    sid = lax.axis_index("s")

    pltpu.sync_copy(es_hbm.at[1, sid], didx)

    def fill(c, _):
        ones_v[pl.ds(c * 16, 16)] = jnp.ones((16,), jnp.float32)
        return 0
    lax.fori_loop(0, K // 16, fill, 0)
    _zero_vmem_1d(obuf, RPT)
    pltpu.sync_copy(obuf, acc.at[pl.ds(sid * RPT, RPT)])
    plsc.subcore_barrier()

    def grp(g, _):
        descs = [
            pltpu.async_copy(
                ones_v, acc.at[didx.at[cid * NCHD + g * KGRP + b]],
                sem, add=True)
            for b in range(KGRP)
        ]
        for d in descs:
            d.wait()
        return 0
    lax.fori_loop(0, NCHD // KGRP, grp, 0)
    plsc.subcore_barrier()

    pltpu.sync_copy(acc.at[pl.ds(sid * RPT, RPT)], obuf)
    pltpu.sync_copy(obuf, out_hbm.at[cid, pl.ds(sid * RPT, RPT)])


@functools.partial(
    pl.kernel,
    out_type=jax.ShapeDtypeStruct((NC, NPAD, 64), jnp.float32),
    mesh=_MESH,
    compiler_params=pltpu.CompilerParams(use_tc_tiling_on_sc=False),
    scratch_types=[
        pltpu.VMEM((BLK, K1), jnp.int32),
        pltpu.VMEM((BLK, K1), jnp.int32),
        pltpu.VMEM((BLK, K1), jnp.int32),
        pltpu.VMEM((BLK, K1), jnp.int32),
        *[pltpu.VMEM((K1, 64), jnp.float32) for _ in range(NSLOT)],
        pltpu.VMEM_SHARED((N, 64), jnp.float32),
        pltpu.VMEM_SHARED((NPAD, 64), jnp.float32),
        pltpu.SemaphoreType.DMA,
        pltpu.SemaphoreType.DMA,
        *[pltpu.SemaphoreType.DMA for _ in range(NSLOT)],
        *[pltpu.SemaphoreType.DMA for _ in range(NSLOT)],
        pltpu.SemaphoreType.DMA,
    ],
)
def _agg128(h0_hbm, h1_hbm, es1_hbm, out_hbm, sb0, sb1, db0, db1,
            r0, r1, r2, r3, r4, r5, r6, r7, htab, acc, i0, i1,
            g0, g1, g2, g3, g4, g5, g6, g7,
            c0, c1, c2, c3, c4, c5, c6, c7, tsem):
    rows = (r0, r1, r2, r3, r4, r5, r6, r7)
    gsem = (g0, g1, g2, g3, g4, g5, g6, g7)
    ssem = (c0, c1, c2, c3, c4, c5, c6, c7)
    sbuf = (sb0, sb1)
    dbuf = (db0, db1)
    isem = (i0, i1)
    cid = lax.axis_index("c")
    sid = lax.axis_index("s")

    def gather(sb, l, b):
        pltpu.async_copy(htab.at[sb.at[l]], rows[b], gsem[b])

    def gwait(sb, l, b):
        pltpu.make_async_copy(htab.at[sb.at[l]], rows[b], gsem[b]).wait()

    def scat(db, l, b):
        pltpu.async_copy(rows[b], acc.at[db.at[l]], ssem[b], add=True)

    def swait(b):
        pltpu.make_async_copy(rows[b], acc.at[dbuf[0].at[0]], ssem[b]).wait()

    @pl.when(cid == 0)
    def _():
        pltpu.async_copy(h0_hbm.at[pl.ds(sid * NRT, NRT)],
                         htab.at[pl.ds(sid * NRT, NRT)], tsem)

    @pl.when(cid != 0)
    def _():
        pltpu.async_copy(h1_hbm.at[pl.ds(sid * NRT, NRT)],
                         htab.at[pl.ds(sid * NRT, NRT)], tsem)

    pltpu.async_copy(es1_hbm.at[0, sid, pl.ds(0, BLK)], sbuf[0], isem[0])
    pltpu.async_copy(es1_hbm.at[1, sid, pl.ds(0, BLK)], dbuf[0], isem[0])

    _zero_vmem_2d(rows[0], 64, 64)

    def zc(c, _):
        pltpu.sync_copy(rows[0], acc.at[pl.ds(sid * RPT + c * 64, 64)])
        return 0
    lax.fori_loop(0, RPT // 64, zc, 0)

    pltpu.make_async_copy(
        es1_hbm.at[0, sid, pl.ds(0, BLK)], sbuf[0], isem[0]).wait()
    pltpu.make_async_copy(
        es1_hbm.at[1, sid, pl.ds(0, BLK)], dbuf[0], isem[0]).wait()
    pltpu.make_async_copy(
        h0_hbm.at[pl.ds(sid * NRT, NRT)], htab.at[pl.ds(sid * NRT, NRT)],
        tsem).wait()
    plsc.subcore_barrier()

    for blk in range(NBLK):
        p = blk % 2
        sb, db = sbuf[p], dbuf[p]

        if blk == 0:
            for b in range(LOOK):
                gather(sb, jnp.int32(b), b)
            for b in range(NSLOT):
                gwait(sb, jnp.int32(b), b)
                scat(db, jnp.int32(b), b)
                if b >= NSLOT - LOOK:
                    swait((b + LOOK) % NSLOT)
                gather(sb, jnp.int32(b + LOOK), (b + LOOK) % NSLOT)
            g_start = 1
        else:
            g_start = 0

        def grp(g, _):
            for b in range(NSLOT):
                l = g * NSLOT + b
                c = (b + LOOK) % NSLOT
                gwait(sb, l, b)
                scat(db, l, b)
                swait(c)
                gather(sb, l + LOOK, c)
            return 0
        lax.fori_loop(g_start, (BLK - NSLOT) // NSLOT, grp, 0)

        if blk + 1 < NBLK:
            pltpu.async_copy(es1_hbm.at[0, sid, pl.ds((blk + 1) * BLK, BLK)],
                             sbuf[1 - p], isem[1 - p])
            pltpu.async_copy(es1_hbm.at[1, sid, pl.ds((blk + 1) * BLK, BLK)],
                             dbuf[1 - p], isem[1 - p])

        for b in range(NSLOT):
            l = BLK - NSLOT + b
            c = (b + LOOK) % NSLOT
            gwait(sb, jnp.int32(l), b)
            scat(db, jnp.int32(l), b)
            swait(c)
            if b < NSLOT - LOOK:
                gather(sb, jnp.int32(l + LOOK), c)
            elif blk + 1 < NBLK:
                if b == NSLOT - LOOK:
                    pltpu.make_async_copy(
                        es1_hbm.at[0, sid, pl.ds((blk + 1) * BLK, BLK)],
                        sbuf[1 - p], isem[1 - p]).wait()
                    pltpu.make_async_copy(
                        es1_hbm.at[1, sid, pl.ds((blk + 1) * BLK, BLK)],
                        dbuf[1 - p], isem[1 - p]).wait()
                gather(sbuf[1 - p], jnp.int32(b - (NSLOT - LOOK)), c)

    for b in range(LOOK, NSLOT):
        swait(b)
    plsc.subcore_barrier()

    def oc(c, _):
        r0_ = sid * RPT + c * 64
        pltpu.sync_copy(acc.at[pl.ds(r0_, 64)], rows[0])
        pltpu.sync_copy(rows[0], out_hbm.at[cid, pl.ds(r0_, 64)])
        return 0
    lax.fori_loop(0, RPT // 64, oc, 0)


@functools.partial(
    pl.kernel,
    out_type=jax.ShapeDtypeStruct((NC, NPAD, 32), jnp.float32),
    mesh=_MESH,
    compiler_params=pltpu.CompilerParams(use_tc_tiling_on_sc=False),
    scratch_types=[
        pltpu.VMEM((NCH1, K1), jnp.int32),
        pltpu.VMEM((NCH1, K1), jnp.int32),
        *[pltpu.VMEM((K1, 32), jnp.float32) for _ in range(NSLOT)],
        pltpu.VMEM((128, 32), jnp.float32),
        pltpu.VMEM_SHARED((N, 32), jnp.float32),
        pltpu.VMEM_SHARED((NPAD, 32), jnp.float32),
        *[pltpu.SemaphoreType.DMA for _ in range(NSLOT)],
        *[pltpu.SemaphoreType.DMA for _ in range(NSLOT)],
        pltpu.SemaphoreType.DMA,
    ],
)
def _agg64(h0_hbm, h1_hbm, es2_hbm, out_hbm, sidx, didx,
           r0, r1, r2, r3, r4, r5, r6, r7,
           obuf, htab, acc,
           g0, g1, g2, g3, g4, g5, g6, g7,
           c0, c1, c2, c3, c4, c5, c6, c7, tsem):
    rows = (r0, r1, r2, r3, r4, r5, r6, r7)
    gsem = (g0, g1, g2, g3, g4, g5, g6, g7)
    ssem = (c0, c1, c2, c3, c4, c5, c6, c7)
    cid = lax.axis_index("c")
    sid = lax.axis_index("s")

    def issue_gather(i, b):
        m = lax.rem(i, 16)

        @pl.when(jnp.logical_and(m < HFRAC, cid == 0))
        def _():
            pltpu.async_copy(h0_hbm.at[sidx.at[i]], rows[b], gsem[b])

        @pl.when(jnp.logical_and(m < HFRAC, cid != 0))
        def _():
            pltpu.async_copy(h1_hbm.at[sidx.at[i]], rows[b], gsem[b])

        @pl.when(m >= HFRAC)
        def _():
            pltpu.async_copy(htab.at[sidx.at[i]], rows[b], gsem[b])

    @pl.when(cid == 0)
    def _():
        pltpu.async_copy(h0_hbm.at[pl.ds(sid * NRT, NRT)],
                         htab.at[pl.ds(sid * NRT, NRT)], tsem)

    @pl.when(cid != 0)
    def _():
        pltpu.async_copy(h1_hbm.at[pl.ds(sid * NRT, NRT)],
                         htab.at[pl.ds(sid * NRT, NRT)], tsem)

    pltpu.sync_copy(es2_hbm.at[0, sid], sidx)
    pltpu.sync_copy(es2_hbm.at[1, sid], didx)

    _zero_vmem_2d(obuf, 128, 32)

    def zc(c, _):
        pltpu.sync_copy(obuf, acc.at[pl.ds(sid * RPT + c * 128, 128)])
        return 0
    lax.fori_loop(0, RPT // 128, zc, 0)

    pltpu.make_async_copy(
        h0_hbm.at[pl.ds(sid * NRT, NRT)], htab.at[pl.ds(sid * NRT, NRT)],
        tsem).wait()
    plsc.subcore_barrier()

    for b in range(LOOK):
        issue_gather(jnp.int32(b), b)

    for b in range(NSLOT):
        c = (b + LOOK) % NSLOT
        pltpu.make_async_copy(htab.at[sidx.at[b]], rows[b], gsem[b]).wait()
        pltpu.async_copy(rows[b], acc.at[didx.at[b]], ssem[b], add=True)
        if b >= NSLOT - LOOK:
            pltpu.make_async_copy(rows[c], acc.at[didx.at[b]], ssem[c]).wait()
        issue_gather(jnp.int32(b + LOOK), c)

    def grp(g, _):
        for b in range(NSLOT):
            i = g * NSLOT + b
            c = (b + LOOK) % NSLOT
            pltpu.make_async_copy(
                htab.at[sidx.at[i]], rows[b], gsem[b]).wait()
            pltpu.async_copy(rows[b], acc.at[didx.at[i]], ssem[b], add=True)
            pltpu.make_async_copy(rows[c], acc.at[didx.at[i]], ssem[c]).wait()

            @pl.when(i + LOOK < NCH1)
            def _():
                issue_gather(i + LOOK, c)
        return 0
    lax.fori_loop(1, NCH1 // NSLOT, grp, 0)
    for b in range(LOOK, NSLOT):
        pltpu.make_async_copy(rows[b], acc.at[didx.at[0]], ssem[b]).wait()
    plsc.subcore_barrier()

    def oc(c, _):
        r0_ = sid * RPT + c * 128
        pltpu.sync_copy(acc.at[pl.ds(r0_, 128)], obuf)
        pltpu.sync_copy(obuf, out_hbm.at[cid, pl.ds(r0_, 128)])
        return 0
    lax.fori_loop(0, RPT // 128, oc, 0)


def _mm0_body(x_ref, w_ref, o_ref):
    o_ref[...] = jnp.dot(x_ref[...], w_ref[...],
                         preferred_element_type=jnp.float32)


def _mm1_body(h_ref, d0_ref, d1_ref, oa_ref, ob_ref):
    dinv = lax.rsqrt(d0_ref[...] + d1_ref[...] + 1.0)
    hp = h_ref[...] * dinv
    oa_ref[...] = hp[:, :64]
    ob_ref[...] = hp[:, 64:]


def _mm2_body(p_ref, ha_ref, hb_ref, d0_ref, d1_ref, b_ref, w_ref,
              oa_ref, ob_ref):
    dinv = lax.rsqrt(d0_ref[...] + d1_ref[...] + 1.0)
    sl = p_ref[0, :N] + ha_ref[...]
    sr = p_ref[1, :N] + hb_ref[...]
    s = jnp.concatenate((sl, sr), axis=1)
    z = jnp.maximum(s * dinv + b_ref[...], 0.0)
    h = jnp.dot(z, w_ref[...], preferred_element_type=jnp.float32)
    hp = h * dinv
    oa_ref[...] = hp[:, :32]
    ob_ref[...] = hp[:, 32:]


def _out_body(q_ref, ha_ref, hb_ref, d0_ref, d1_ref, b_ref, o_ref):
    dinv = lax.rsqrt(d0_ref[...] + d1_ref[...] + 1.0)
    sl = q_ref[0, :N] + ha_ref[...]
    sr = q_ref[1, :N] + hb_ref[...]
    s = jnp.concatenate((sl, sr), axis=1)
    o_ref[...] = s * dinv + b_ref[...]


_mm0 = pl.pallas_call(
    _mm0_body, out_shape=jax.ShapeDtypeStruct((N, 128), jnp.float32))
_mm1 = pl.pallas_call(
    _mm1_body,
    out_shape=(jax.ShapeDtypeStruct((N, 64), jnp.float32),
               jax.ShapeDtypeStruct((N, 64), jnp.float32)))
_mm2 = pl.pallas_call(
    _mm2_body,
    out_shape=(jax.ShapeDtypeStruct((N, 32), jnp.float32),
               jax.ShapeDtypeStruct((N, 32), jnp.float32)))
_mm3 = pl.pallas_call(
    _out_body, out_shape=jax.ShapeDtypeStruct((N, 64), jnp.float32))


def kernel(x, edge_index, W1, b1, W2, b2):
    pad = EPAD - E
    sd = jnp.concatenate(
        [edge_index,
         jnp.stack([jnp.zeros((pad,), jnp.int32),
                    jnp.full((pad,), NPAD - 1, jnp.int32)])], axis=1)
    es1 = sd.reshape(2, NS, NCH1, K1)
    es2 = sd.reshape(2, NS, NCH2, K)

    h1 = _mm0(x, W1)
    degp = _deg_kernel(es2)
    d0 = degp[0, :N].reshape(N, 1)
    d1 = degp[1, :N].reshape(N, 1)

    h1a, h1b = _mm1(h1, d0, d1)
    p = _agg128(h1a, h1b, es1)
    h2a, h2b = _mm2(p, h1a, h1b, d0, d1, b1.reshape(1, 128), W2)
    q = _agg64(h2a, h2b, es1)
    return _mm3(q, h2a, h2b, d0, d1, b2.reshape(1, 64))
